# Initial kernel scaffold; baseline (speedup 1.0000x reference)
import jax
import jax.numpy as jnp
from jax.experimental import pallas as pl


def kernel(x, edge_index, batch, W1, b1, W2, b2, Wc, bc):
    raise NotImplementedError("write your pallas kernel here")


# speedup vs baseline: 14.9271x; 14.9271x over previous
import functools

import jax
import jax.numpy as jnp
from jax import lax
from jax.experimental import pallas as pl
from jax.experimental.pallas import tpu as pltpu
from jax.experimental.pallas import tpu_sc as plsc

N = 10000
E = 320000
D = 128
H = 64
C = 16
G = 64

NC = 2
NS = 16
NW = NC * NS
EPW = E // NW
BLK = 80
NBLK = EPW // BLK
N_PAD = 10240
RPW = N_PAD // NS
DEGW = 16

assert E % NW == 0 and EPW % BLK == 0 and N_PAD % (8 * NS) == 0

_MESH = plsc.VectorSubcoreMesh(
    core_axis_name="c", subcore_axis_name="s", num_cores=NC, num_subcores=NS)


def _sc_deg_body(dst_hbm, ones_hbm, zero_hbm, out_hbm, dstb, onesv, acc, sem):
    c = lax.axis_index("c")
    s = lax.axis_index("s")
    gwid = c * NS + s
    pltpu.sync_copy(ones_hbm, onesv)
    pltpu.sync_copy(zero_hbm.at[pl.ds(s * RPW, RPW)],
                    acc.at[pl.ds(s * RPW, RPW)])
    plsc.subcore_barrier()

    def body(j, carry):
        off = gwid * EPW + j * BLK
        pltpu.sync_copy(dst_hbm.at[pl.ds(off, BLK)], dstb)
        pltpu.sync_copy(onesv, acc.at[dstb], add=True)
        return carry

    lax.fori_loop(0, NBLK, body, 0)
    plsc.subcore_barrier()
    pltpu.sync_copy(acc.at[pl.ds(s * RPW, RPW)],
                    out_hbm.at[pl.ds(c * N_PAD + s * RPW, RPW)])


_SC_PARAMS = pltpu.CompilerParams(use_tc_tiling_on_sc=False)

_sc_degree = functools.partial(
    pl.kernel,
    out_type=jax.ShapeDtypeStruct((NC * N_PAD, DEGW), jnp.float32),
    mesh=_MESH,
    compiler_params=_SC_PARAMS,
    scratch_types=[
        pltpu.VMEM((BLK,), jnp.int32),
        pltpu.VMEM((BLK, DEGW), jnp.float32),
        pltpu.VMEM_SHARED((N_PAD, DEGW), jnp.float32),
        pltpu.SemaphoreType.DMA,
    ],
)(_sc_deg_body)


def _sc_conv_body(g_hbm, src_hbm, dst_hbm, zero_hbm, out_hbm,
                  srcb, dstb, rows, acc, sem):
    c = lax.axis_index("c")
    s = lax.axis_index("s")
    gwid = c * NS + s
    pltpu.sync_copy(zero_hbm.at[pl.ds(s * RPW, RPW)],
                    acc.at[pl.ds(s * RPW, RPW)])
    plsc.subcore_barrier()

    def body(j, carry):
        off = gwid * EPW + j * BLK
        pltpu.sync_copy(src_hbm.at[pl.ds(off, BLK)], srcb)
        pltpu.sync_copy(dst_hbm.at[pl.ds(off, BLK)], dstb)
        pltpu.async_copy(g_hbm.at[srcb], rows, sem).wait()
        pltpu.sync_copy(rows, acc.at[dstb], add=True)
        return carry

    lax.fori_loop(0, NBLK, body, 0)
    plsc.subcore_barrier()
    pltpu.sync_copy(acc.at[pl.ds(s * RPW, RPW)],
                    out_hbm.at[pl.ds(c * N_PAD + s * RPW, RPW)])


_sc_conv = functools.partial(
    pl.kernel,
    out_type=jax.ShapeDtypeStruct((NC * N_PAD, H), jnp.float32),
    mesh=_MESH,
    compiler_params=_SC_PARAMS,
    scratch_types=[
        pltpu.VMEM((BLK,), jnp.int32),
        pltpu.VMEM((BLK,), jnp.int32),
        pltpu.VMEM((BLK, H), jnp.float32),
        pltpu.VMEM_SHARED((N_PAD, H), jnp.float32),
        pltpu.SemaphoreType.DMA,
    ],
)(_sc_conv_body)


def _tc_prep_body(x_ref, w1_ref, degp_ref, g1_ref, dinv_ref):
    deg = degp_ref[0:N, 0:1] + degp_ref[N_PAD:N_PAD + N, 0:1] + 1.0
    dinv = lax.rsqrt(deg)
    h1 = jnp.dot(x_ref[...], w1_ref[...], preferred_element_type=jnp.float32)
    dinv_ref[...] = dinv
    g1_ref[...] = h1 * dinv


def _tc_prep(x, w1, degp):
    return pl.pallas_call(
        _tc_prep_body,
        out_shape=(jax.ShapeDtypeStruct((N, H), jnp.float32),
                   jax.ShapeDtypeStruct((N, 1), jnp.float32)),
    )(x, w1, degp)


def _tc_mid_body(sp_ref, g1_ref, dinv_ref, b1_ref, w2_ref, g2_ref):
    ssum = sp_ref[0:N, :] + sp_ref[N_PAD:N_PAD + N, :]
    dinv = dinv_ref[...]
    h1 = jnp.maximum(dinv * (ssum + g1_ref[...]) + b1_ref[...], 0.0)
    h2 = jnp.dot(h1, w2_ref[...], preferred_element_type=jnp.float32)
    g2_ref[...] = h2 * dinv


def _tc_mid(sp, g1, dinv, b1, w2):
    return pl.pallas_call(
        _tc_mid_body,
        out_shape=jax.ShapeDtypeStruct((N, H), jnp.float32),
    )(sp, g1, dinv, b1, w2)


def _tc_final_body(sp_ref, g2_ref, dinv_ref, b2_ref, batch_ref, wc_ref,
                   bc_ref, out_ref):
    ssum = sp_ref[0:N, :] + sp_ref[N_PAD:N_PAD + N, :]
    z = jnp.maximum(dinv_ref[...] * (ssum + g2_ref[...]) + b2_ref[...], 0.0)
    gids = lax.broadcasted_iota(jnp.int32, (G, N), 0)
    sel = (gids == batch_ref[...]).astype(jnp.float32)
    sums = jnp.dot(sel, z, preferred_element_type=jnp.float32)
    counts = jnp.sum(sel, axis=1, keepdims=True)
    pooled = sums / jnp.maximum(counts, 1.0)
    out_ref[...] = (
        jnp.dot(pooled, wc_ref[...], preferred_element_type=jnp.float32)
        + bc_ref[...])


def _tc_final(sp, g2, dinv, b2, batch2d, wc, bc):
    return pl.pallas_call(
        _tc_final_body,
        out_shape=jax.ShapeDtypeStruct((G, C), jnp.float32),
    )(sp, g2, dinv, b2, batch2d, wc, bc)


def kernel(x, edge_index, batch, W1, b1, W2, b2, Wc, bc):
    src = edge_index[0].astype(jnp.int32)
    dst = edge_index[1].astype(jnp.int32)
    batch2d = batch.astype(jnp.int32).reshape(1, N)
    ones_blk = jnp.ones((BLK, DEGW), jnp.float32)
    zeros_deg = jnp.zeros((N_PAD, DEGW), jnp.float32)
    zeros_h = jnp.zeros((N_PAD, H), jnp.float32)

    degp = _sc_degree(dst, ones_blk, zeros_deg)
    g1, dinv = _tc_prep(x, W1, degp)
    s1 = _sc_conv(g1, src, dst, zeros_h)
    g2 = _tc_mid(s1, g1, dinv, b1.reshape(1, H), W2)
    s2 = _sc_conv(g2, src, dst, zeros_h)
    return _tc_final(s2, g2, dinv, b2.reshape(1, H), batch2d, Wc,
                     bc.reshape(1, C))
